# baseline (speedup 1.0000x reference)
import functools

import jax
import jax.numpy as jnp
from jax import lax
from jax.experimental import pallas as pl
from jax.experimental.pallas import tpu as pltpu
from jax.experimental.pallas import tpu_sc as plsc

S = 2048
D = 1024
FF = 4096
E = 8
K = 2
TILE = 256
NT = 24
NPAD = NT * TILE
TMAX = S // TILE
FC = 1024
NF = FF // FC
CH = 256
NW = 32
TPW = S // NW
CHT = 32


def _router_body(x_ref, wg_ref, bg_ref, pos_ref, pos2_ref, ws_ref, meta_ref,
                 aux_ref):
    x = x_ref[...]
    logits = jnp.dot(x, wg_ref[...], preferred_element_type=jnp.float32)
    logits = logits + bg_ref[...]
    m = jnp.max(logits, axis=1, keepdims=True)
    ex = jnp.exp(logits - m)
    p = ex / jnp.sum(ex, axis=1, keepdims=True)
    idx = lax.broadcasted_iota(jnp.int32, (S, E), 1)
    v1k = jnp.max(p, axis=1, keepdims=True)
    e1 = jnp.min(jnp.where(p >= v1k, idx, E), axis=1, keepdims=True)
    o1 = idx == e1
    masked = jnp.where(o1, -1.0, p)
    v2k = jnp.max(masked, axis=1, keepdims=True)
    e2 = jnp.min(jnp.where(masked >= v2k, idx, E), axis=1, keepdims=True)
    o2 = idx == e2
    O1f = o1.astype(jnp.float32)
    O2f = o2.astype(jnp.float32)
    counts = jnp.sum(O1f, axis=0, keepdims=True) + jnp.sum(O2f, axis=0,
                                                           keepdims=True)
    P_i = jnp.sum(p, axis=0, keepdims=True) * (1.0 / S)
    f_i = counts * (1.0 / (S * K))
    aux_ref[...] = E * jnp.sum(f_i * P_i, axis=1, keepdims=True)
    tiles = jnp.ceil(counts * (1.0 / TILE))
    r8 = lax.broadcasted_iota(jnp.int32, (E, E), 0)
    c8 = lax.broadcasted_iota(jnp.int32, (E, E), 1)
    upper = (r8 < c8).astype(jnp.float32)
    st_tiles = jnp.dot(tiles, upper, preferred_element_type=jnp.float32)
    st_rows = st_tiles * TILE
    cum_tiles = (st_tiles + tiles).astype(jnp.int32)
    ttio = lax.broadcasted_iota(jnp.int32, (NT, E), 0)
    owner_k = jnp.sum((jnp.broadcast_to(cum_tiles, (NT, E)) <= ttio)
                      .astype(jnp.int32), axis=1, keepdims=True)
    owner_k = jnp.minimum(owner_k, E - 1)
    meta_ref[pl.ds(0, NT)] = owner_k.reshape(NT)
    meta_ref[pl.ds(NT, 1)] = jnp.sum(tiles, axis=1).astype(jnp.int32)
    act = (tiles >= 1.0).astype(jnp.float32)
    rank_excl = jnp.dot(act, upper, preferred_element_type=jnp.float32)
    iota_e_nt = lax.broadcasted_iota(jnp.int32, (NT, E), 1)
    oh_own = (iota_e_nt == owner_k).astype(jnp.float32)
    ordv = jnp.sum(oh_own * jnp.broadcast_to(rank_excl, (NT, E)), axis=1)
    meta_ref[pl.ds(NT + 1, NT)] = ordv.astype(jnp.int32)
    meta_ref[pl.ds(2 * NT + 1, 1)] = jnp.sum(act, axis=1).astype(jnp.int32)
    rr8f = lax.broadcasted_iota(jnp.int32, (E, E), 0).astype(jnp.float32)
    ee8f = lax.broadcasted_iota(jnp.int32, (E, E), 1).astype(jnp.float32)
    sel = (jnp.broadcast_to(rank_excl, (E, E)) == rr8f).astype(jnp.float32)
    blk_e = jnp.sum(sel * jnp.broadcast_to(act, (E, E)) * ee8f, axis=1)
    meta_ref[pl.ds(2 * NT + 2, E)] = blk_e.astype(jnp.int32)
    rr = lax.broadcasted_iota(jnp.int32, (CH, CH), 0)
    cc = lax.broadcasted_iota(jnp.int32, (CH, CH), 1)
    tri = (rr >= cc).astype(jnp.float32)
    run = jnp.zeros((1, E), jnp.float32)
    for half, of in enumerate((O1f, O2f)):
        base = half * S
        for c in range(S // CH):
            oc = of[c * CH:(c + 1) * CH, :]
            pc = jnp.dot(tri, oc, preferred_element_type=jnp.float32) + run
            run = run + jnp.sum(oc, axis=0, keepdims=True)
            posck = jnp.sum(oc * (st_rows + pc - 1.0), axis=1, keepdims=True)
            pos_ref[pl.ds(base + c * CH, CH)] = (
                posck.reshape(CH).astype(jnp.int32))
            pos2_ref[pl.ds(c * CH, CH), half:half + 1] = posck.astype(jnp.int32)
    ws_ref[pl.ds(0, S), :] = jnp.broadcast_to(v1k, (S, 16))
    ws_ref[pl.ds(S, S), :] = jnp.broadcast_to(v2k, (S, 16))


def _router(xf, wg, bg2):
    return pl.pallas_call(
        _router_body,
        out_shape=(
            jax.ShapeDtypeStruct((K * S,), jnp.int32),
            jax.ShapeDtypeStruct((S, K), jnp.int32),
            jax.ShapeDtypeStruct((K * S, 16), jnp.float32),
            jax.ShapeDtypeStruct((2 * NT + 2 + E,), jnp.int32),
            jax.ShapeDtypeStruct((1, 1), jnp.float32),
        ),
    )(xf, wg, bg2)


@functools.cache
def _sc_dispatch():
    mesh = plsc.VectorSubcoreMesh(core_axis_name="c", subcore_axis_name="s")

    @functools.partial(
        pl.kernel,
        out_type=jax.ShapeDtypeStruct((NPAD, D), jnp.float32),
        mesh=mesh,
        scratch_types=[
            pltpu.VMEM((TPW, D), jnp.float32),
            pltpu.VMEM((TPW,), jnp.int32),
            pltpu.VMEM((TPW,), jnp.int32),
            pltpu.SemaphoreType.DMA,
            pltpu.SemaphoreType.DMA,
        ],
    )
    def _dispatch(x_hbm, pos_hbm, xs_hbm, xrows, idx0, idx1, sem0, sem1):
        wid = lax.axis_index("s") * mesh.num_cores + lax.axis_index("c")
        base = wid * TPW
        pltpu.sync_copy(x_hbm.at[pl.ds(base, TPW)], xrows)
        pltpu.sync_copy(pos_hbm.at[pl.ds(base, TPW)], idx0)
        pltpu.sync_copy(pos_hbm.at[pl.ds(S + base, TPW)], idx1)
        c0 = pltpu.async_copy(xrows, xs_hbm.at[idx0], sem0)
        c1 = pltpu.async_copy(xrows, xs_hbm.at[idx1], sem1)
        c0.wait()
        c1.wait()

    return _dispatch


OFF_ORD = NT + 1
OFF_NA = 2 * NT + 1
OFF_BLK = 2 * NT + 2


def _experts_body(meta_ref, xs_ref, w1_ref, w3_ref, w2_ref, out_ref,
                  acc_ref, w1f_ref, w3f_ref, w2f_ref,
                  w1c_ref, w3c_ref, w2c_ref, sem1, sem3, sem2):
    f = pl.program_id(0)
    tt = pl.program_id(1)
    na = meta_ref[OFF_NA]

    def copies(blk, slot):
        e_n = meta_ref[OFF_BLK + lax.rem(blk, na)]
        f_n = blk // na
        c1 = pltpu.make_async_copy(
            w1_ref.at[e_n, :, pl.ds(f_n * FC, FC)], w1f_ref.at[slot],
            sem1.at[slot])
        c3 = pltpu.make_async_copy(
            w3_ref.at[e_n, :, pl.ds(f_n * FC, FC)], w3f_ref.at[slot],
            sem3.at[slot])
        c2 = pltpu.make_async_copy(
            w2_ref.at[e_n, pl.ds(f_n * FC, FC), :], w2f_ref.at[slot],
            sem2.at[slot])
        return c1, c3, c2

    @pl.when(tt < meta_ref[NT])
    def _():
        b = f * na + meta_ref[OFF_ORD + tt]
        slot = lax.rem(b, 2)
        nblk = NF * na
        is_new = (tt == 0) | (meta_ref[tt] != meta_ref[jnp.maximum(tt - 1, 0)])

        @pl.when((f == 0) & (tt == 0))
        def _():
            for c in copies(b, slot):
                c.start()

        @pl.when(is_new)
        def _():
            @pl.when(b + 1 < nblk)
            def _():
                for c in copies(b + 1, 1 - slot):
                    c.start()

            for c in copies(b, slot):
                c.wait()
            w1c_ref[...] = w1f_ref[slot].astype(jnp.bfloat16)
            w3c_ref[...] = w3f_ref[slot].astype(jnp.bfloat16)
            w2c_ref[...] = w2f_ref[slot].astype(jnp.bfloat16)

        a = xs_ref[...].astype(jnp.bfloat16)
        g = jnp.dot(a, w1c_ref[...], preferred_element_type=jnp.float32)
        u = jnp.dot(a, w3c_ref[...], preferred_element_type=jnp.float32)
        h = (g * lax.logistic(g) * u).astype(jnp.bfloat16)
        part = jnp.dot(h, w2c_ref[...], preferred_element_type=jnp.float32)
        sl = pl.ds(tt * TILE, TILE)

        @pl.when(f == 0)
        def _():
            acc_ref[sl, :] = part.astype(jnp.bfloat16)

        @pl.when((f > 0) & (f < NF - 1))
        def _():
            acc_ref[sl, :] = (acc_ref[sl, :].astype(jnp.float32)
                              + part).astype(jnp.bfloat16)

        @pl.when(f == NF - 1)
        def _():
            out_ref[...] = acc_ref[sl, :].astype(jnp.float32) + part


def _experts(meta, xs, w1, w3, w2):
    grid_spec = pltpu.PrefetchScalarGridSpec(
        num_scalar_prefetch=1,
        grid=(NF, NT),
        in_specs=[
            pl.BlockSpec((TILE, D), lambda f, tt, m: (tt, 0)),
            pl.BlockSpec(memory_space=pl.ANY),
            pl.BlockSpec(memory_space=pl.ANY),
            pl.BlockSpec(memory_space=pl.ANY),
        ],
        out_specs=pl.BlockSpec((TILE, D), lambda f, tt, m: (tt, 0)),
        scratch_shapes=[
            pltpu.VMEM((NPAD, D), jnp.bfloat16),
            pltpu.VMEM((2, D, FC), jnp.float32),
            pltpu.VMEM((2, D, FC), jnp.float32),
            pltpu.VMEM((2, FC, D), jnp.float32),
            pltpu.VMEM((D, FC), jnp.bfloat16),
            pltpu.VMEM((D, FC), jnp.bfloat16),
            pltpu.VMEM((FC, D), jnp.bfloat16),
            pltpu.SemaphoreType.DMA((2,)),
            pltpu.SemaphoreType.DMA((2,)),
            pltpu.SemaphoreType.DMA((2,)),
        ],
    )
    return pl.pallas_call(
        _experts_body,
        grid_spec=grid_spec,
        out_shape=jax.ShapeDtypeStruct((NPAD, D), jnp.float32),
    )(meta, xs, w1, w3, w2)


@functools.cache
def _sc_combine():
    mesh = plsc.VectorSubcoreMesh(core_axis_name="c", subcore_axis_name="s")

    @functools.partial(
        pl.kernel,
        out_type=jax.ShapeDtypeStruct((S, D), jnp.float32),
        mesh=mesh,
        scratch_types=[
            pltpu.VMEM((2 * CHT, D), jnp.float32),
            pltpu.VMEM((CHT, D), jnp.float32),
            pltpu.VMEM((2 * CHT,), jnp.int32),
            pltpu.VMEM((CHT, 16), jnp.float32),
            pltpu.VMEM((CHT, 16), jnp.float32),
            pltpu.SemaphoreType.DMA,
        ],
    )
    def _combine(ys_hbm, pos2_hbm, ws_hbm, out_hbm, buf, ob, idx2,
                 w0, w1, sem):
        wid = lax.axis_index("s") * mesh.num_cores + lax.axis_index("c")
        for chunk in range(TPW // CHT):
            base = wid * TPW + chunk * CHT
            pltpu.sync_copy(pos2_hbm.at[pl.ds(2 * base, 2 * CHT)], idx2)
            pltpu.sync_copy(ws_hbm.at[pl.ds(base, CHT)], w0)
            pltpu.sync_copy(ws_hbm.at[pl.ds(S + base, CHT)], w1)
            pltpu.async_copy(ys_hbm.at[idx2], buf, sem).wait()

            def row_body(r, carry):
                wa = w0[r, :]
                wb = w1[r, :]

                def col_body(ci, carry2):
                    sl = pl.ds(ci * 16, 16)
                    ob[r, sl] = wa * buf[2 * r, sl] + wb * buf[2 * r + 1, sl]
                    return carry2

                return lax.fori_loop(0, D // 16, col_body, carry)

            lax.fori_loop(0, CHT, row_body, 0)
            pltpu.sync_copy(ob, out_hbm.at[pl.ds(base, CHT)])

    return _combine


def kernel(x, Wg, bg, W1, W3, W2):
    xf = x.reshape(S, D)
    pos, pos2, ws, meta, aux = _router(xf, Wg, bg.reshape(1, E))
    xs = _sc_dispatch()(xf, pos)
    ys = _experts(meta, xs, W1, W3, W2)
    out = _sc_combine()(ys, pos2.reshape(K * S), ws)
    return (out.reshape(x.shape), aux[0, 0])


# speedup vs baseline: 1.0095x; 1.0095x over previous
import functools

import jax
import jax.numpy as jnp
from jax import lax
from jax.experimental import pallas as pl
from jax.experimental.pallas import tpu as pltpu
from jax.experimental.pallas import tpu_sc as plsc

S = 2048
D = 1024
FF = 4096
E = 8
K = 2
TILE = 256
NT = 24
NPAD = NT * TILE
TMAX = S // TILE
FC = 1024
NF = FF // FC
CH = 256
NW = 32
TPW = S // NW
CHT = 32


def _router_body(x_ref, wg_ref, bg_ref, pos_ref, ws_ref, meta_ref, aux_ref):
    x = x_ref[...]
    logits = jnp.dot(x, wg_ref[...], preferred_element_type=jnp.float32)
    logits = logits + bg_ref[...]
    m = jnp.max(logits, axis=1, keepdims=True)
    ex = jnp.exp(logits - m)
    p = ex / jnp.sum(ex, axis=1, keepdims=True)
    idx = lax.broadcasted_iota(jnp.int32, (S, E), 1)
    v1k = jnp.max(p, axis=1, keepdims=True)
    e1 = jnp.min(jnp.where(p >= v1k, idx, E), axis=1, keepdims=True)
    o1 = idx == e1
    masked = jnp.where(o1, -1.0, p)
    v2k = jnp.max(masked, axis=1, keepdims=True)
    e2 = jnp.min(jnp.where(masked >= v2k, idx, E), axis=1, keepdims=True)
    o2 = idx == e2
    O1f = o1.astype(jnp.float32)
    O2f = o2.astype(jnp.float32)
    counts = jnp.sum(O1f, axis=0, keepdims=True) + jnp.sum(O2f, axis=0,
                                                           keepdims=True)
    P_i = jnp.sum(p, axis=0, keepdims=True) * (1.0 / S)
    f_i = counts * (1.0 / (S * K))
    aux_ref[...] = E * jnp.sum(f_i * P_i, axis=1, keepdims=True)
    tiles = jnp.ceil(counts * (1.0 / TILE))
    r8 = lax.broadcasted_iota(jnp.int32, (E, E), 0)
    c8 = lax.broadcasted_iota(jnp.int32, (E, E), 1)
    upper = (r8 < c8).astype(jnp.float32)
    st_tiles = jnp.dot(tiles, upper, preferred_element_type=jnp.float32)
    st_rows = st_tiles * TILE
    cum_tiles = (st_tiles + tiles).astype(jnp.int32)
    ttio = lax.broadcasted_iota(jnp.int32, (NT, E), 0)
    owner_k = jnp.sum((jnp.broadcast_to(cum_tiles, (NT, E)) <= ttio)
                      .astype(jnp.int32), axis=1, keepdims=True)
    owner_k = jnp.minimum(owner_k, E - 1)
    meta_ref[pl.ds(0, NT)] = owner_k.reshape(NT)
    meta_ref[pl.ds(NT, 1)] = jnp.sum(tiles, axis=1).astype(jnp.int32)
    act = (tiles >= 1.0).astype(jnp.float32)
    rank_excl = jnp.dot(act, upper, preferred_element_type=jnp.float32)
    iota_e_nt = lax.broadcasted_iota(jnp.int32, (NT, E), 1)
    oh_own = (iota_e_nt == owner_k).astype(jnp.float32)
    ordv = jnp.sum(oh_own * jnp.broadcast_to(rank_excl, (NT, E)), axis=1)
    meta_ref[pl.ds(NT + 1, NT)] = ordv.astype(jnp.int32)
    meta_ref[pl.ds(2 * NT + 1, 1)] = jnp.sum(act, axis=1).astype(jnp.int32)
    rr8f = lax.broadcasted_iota(jnp.int32, (E, E), 0).astype(jnp.float32)
    ee8f = lax.broadcasted_iota(jnp.int32, (E, E), 1).astype(jnp.float32)
    sel = (jnp.broadcast_to(rank_excl, (E, E)) == rr8f).astype(jnp.float32)
    blk_e = jnp.sum(sel * jnp.broadcast_to(act, (E, E)) * ee8f, axis=1)
    meta_ref[pl.ds(2 * NT + 2, E)] = blk_e.astype(jnp.int32)
    rr = lax.broadcasted_iota(jnp.int32, (CH, CH), 0)
    cc = lax.broadcasted_iota(jnp.int32, (CH, CH), 1)
    tri = (rr >= cc).astype(jnp.float32)
    run = jnp.zeros((1, E), jnp.float32)
    for half, of in enumerate((O1f, O2f)):
        base = half * S
        for c in range(S // CH):
            oc = of[c * CH:(c + 1) * CH, :]
            pc = jnp.dot(tri, oc, preferred_element_type=jnp.float32) + run
            run = run + jnp.sum(oc, axis=0, keepdims=True)
            posc = jnp.sum(oc * (st_rows + pc - 1.0), axis=1)
            pos_ref[pl.ds(base + c * CH, CH)] = posc.astype(jnp.int32)
    ws_ref[pl.ds(0, S), :] = jnp.broadcast_to(v1k, (S, 16))
    ws_ref[pl.ds(S, S), :] = jnp.broadcast_to(v2k, (S, 16))


def _router(xf, wg, bg2):
    return pl.pallas_call(
        _router_body,
        out_shape=(
            jax.ShapeDtypeStruct((K * S,), jnp.int32),
            jax.ShapeDtypeStruct((K * S, 16), jnp.float32),
            jax.ShapeDtypeStruct((2 * NT + 2 + E,), jnp.int32),
            jax.ShapeDtypeStruct((1, 1), jnp.float32),
        ),
    )(xf, wg, bg2)


@functools.cache
def _sc_dispatch():
    mesh = plsc.VectorSubcoreMesh(core_axis_name="c", subcore_axis_name="s")

    @functools.partial(
        pl.kernel,
        out_type=jax.ShapeDtypeStruct((NPAD, D), jnp.float32),
        mesh=mesh,
        scratch_types=[
            pltpu.VMEM((TPW, D), jnp.float32),
            pltpu.VMEM((TPW,), jnp.int32),
            pltpu.VMEM((TPW,), jnp.int32),
            pltpu.SemaphoreType.DMA,
            pltpu.SemaphoreType.DMA,
        ],
    )
    def _dispatch(x_hbm, pos_hbm, xs_hbm, xrows, idx0, idx1, sem0, sem1):
        wid = lax.axis_index("s") * mesh.num_cores + lax.axis_index("c")
        base = wid * TPW
        pltpu.sync_copy(x_hbm.at[pl.ds(base, TPW)], xrows)
        pltpu.sync_copy(pos_hbm.at[pl.ds(base, TPW)], idx0)
        pltpu.sync_copy(pos_hbm.at[pl.ds(S + base, TPW)], idx1)
        c0 = pltpu.async_copy(xrows, xs_hbm.at[idx0], sem0)
        c1 = pltpu.async_copy(xrows, xs_hbm.at[idx1], sem1)
        c0.wait()
        c1.wait()

    return _dispatch


OFF_ORD = NT + 1
OFF_NA = 2 * NT + 1
OFF_BLK = 2 * NT + 2


def _experts_body(meta_ref, xs_ref, w1_ref, w3_ref, w2_ref, out_ref,
                  acc_ref, w1f_ref, w3f_ref, w2f_ref,
                  w1c_ref, w3c_ref, w2c_ref, sem1, sem3, sem2):
    f = pl.program_id(0)
    tt = pl.program_id(1)
    na = meta_ref[OFF_NA]

    def copies(blk, slot):
        e_n = meta_ref[OFF_BLK + lax.rem(blk, na)]
        f_n = blk // na
        c1 = pltpu.make_async_copy(
            w1_ref.at[e_n, :, pl.ds(f_n * FC, FC)], w1f_ref.at[slot],
            sem1.at[slot])
        c3 = pltpu.make_async_copy(
            w3_ref.at[e_n, :, pl.ds(f_n * FC, FC)], w3f_ref.at[slot],
            sem3.at[slot])
        c2 = pltpu.make_async_copy(
            w2_ref.at[e_n, pl.ds(f_n * FC, FC), :], w2f_ref.at[slot],
            sem2.at[slot])
        return c1, c3, c2

    @pl.when(tt < meta_ref[NT])
    def _():
        b = f * na + meta_ref[OFF_ORD + tt]
        slot = lax.rem(b, 2)
        nblk = NF * na
        is_new = (tt == 0) | (meta_ref[tt] != meta_ref[jnp.maximum(tt - 1, 0)])

        @pl.when((f == 0) & (tt == 0))
        def _():
            for c in copies(b, slot):
                c.start()

        @pl.when(is_new)
        def _():
            @pl.when(b + 1 < nblk)
            def _():
                for c in copies(b + 1, 1 - slot):
                    c.start()

            for c in copies(b, slot):
                c.wait()
            w1c_ref[...] = w1f_ref[slot].astype(jnp.bfloat16)
            w3c_ref[...] = w3f_ref[slot].astype(jnp.bfloat16)
            w2c_ref[...] = w2f_ref[slot].astype(jnp.bfloat16)

        a = xs_ref[...].astype(jnp.bfloat16)
        g = jnp.dot(a, w1c_ref[...], preferred_element_type=jnp.float32)
        u = jnp.dot(a, w3c_ref[...], preferred_element_type=jnp.float32)
        h = (g * lax.logistic(g) * u).astype(jnp.bfloat16)
        part = jnp.dot(h, w2c_ref[...], preferred_element_type=jnp.float32)
        sl = pl.ds(tt * TILE, TILE)

        @pl.when(f == 0)
        def _():
            acc_ref[sl, :] = part.astype(jnp.bfloat16)

        @pl.when((f > 0) & (f < NF - 1))
        def _():
            acc_ref[sl, :] = (acc_ref[sl, :].astype(jnp.float32)
                              + part).astype(jnp.bfloat16)

        @pl.when(f == NF - 1)
        def _():
            out_ref[...] = acc_ref[sl, :].astype(jnp.float32) + part


def _experts(meta, xs, w1, w3, w2):
    grid_spec = pltpu.PrefetchScalarGridSpec(
        num_scalar_prefetch=1,
        grid=(NF, NT),
        in_specs=[
            pl.BlockSpec((TILE, D), lambda f, tt, m: (tt, 0)),
            pl.BlockSpec(memory_space=pl.ANY),
            pl.BlockSpec(memory_space=pl.ANY),
            pl.BlockSpec(memory_space=pl.ANY),
        ],
        out_specs=pl.BlockSpec((TILE, D), lambda f, tt, m: (tt, 0)),
        scratch_shapes=[
            pltpu.VMEM((NPAD, D), jnp.bfloat16),
            pltpu.VMEM((2, D, FC), jnp.float32),
            pltpu.VMEM((2, D, FC), jnp.float32),
            pltpu.VMEM((2, FC, D), jnp.float32),
            pltpu.VMEM((D, FC), jnp.bfloat16),
            pltpu.VMEM((D, FC), jnp.bfloat16),
            pltpu.VMEM((FC, D), jnp.bfloat16),
            pltpu.SemaphoreType.DMA((2,)),
            pltpu.SemaphoreType.DMA((2,)),
            pltpu.SemaphoreType.DMA((2,)),
        ],
    )
    return pl.pallas_call(
        _experts_body,
        grid_spec=grid_spec,
        out_shape=jax.ShapeDtypeStruct((NPAD, D), jnp.float32),
    )(meta, xs, w1, w3, w2)


@functools.cache
def _sc_combine():
    mesh = plsc.VectorSubcoreMesh(core_axis_name="c", subcore_axis_name="s")

    @functools.partial(
        pl.kernel,
        out_type=jax.ShapeDtypeStruct((S, D), jnp.float32),
        mesh=mesh,
        scratch_types=[
            pltpu.VMEM((CHT, D), jnp.float32),
            pltpu.VMEM((CHT, D), jnp.float32),
            pltpu.VMEM((CHT,), jnp.int32),
            pltpu.VMEM((CHT,), jnp.int32),
            pltpu.VMEM((CHT, 16), jnp.float32),
            pltpu.VMEM((CHT, 16), jnp.float32),
            pltpu.SemaphoreType.DMA,
            pltpu.SemaphoreType.DMA,
        ],
    )
    def _combine(ys_hbm, pos_hbm, ws_hbm, out_hbm, rows0, rows1, idx0, idx1,
                 w0, w1, sem0, sem1):
        wid = lax.axis_index("s") * mesh.num_cores + lax.axis_index("c")
        for chunk in range(TPW // CHT):
            base = wid * TPW + chunk * CHT
            pltpu.sync_copy(pos_hbm.at[pl.ds(base, CHT)], idx0)
            pltpu.sync_copy(pos_hbm.at[pl.ds(S + base, CHT)], idx1)
            pltpu.sync_copy(ws_hbm.at[pl.ds(base, CHT)], w0)
            pltpu.sync_copy(ws_hbm.at[pl.ds(S + base, CHT)], w1)

            c0 = pltpu.async_copy(ys_hbm.at[idx0], rows0, sem0)
            c1 = pltpu.async_copy(ys_hbm.at[idx1], rows1, sem1)
            c0.wait()
            c1.wait()

            def row_body(r, carry):
                wa = w0[r, :]
                wb = w1[r, :]

                def col_body(ci, carry2):
                    sl = pl.ds(ci * 16, 16)
                    rows0[r, sl] = wa * rows0[r, sl] + wb * rows1[r, sl]
                    return carry2

                return lax.fori_loop(0, D // 16, col_body, carry)

            lax.fori_loop(0, CHT, row_body, 0)
            pltpu.sync_copy(rows0, out_hbm.at[pl.ds(base, CHT)])

    return _combine


def kernel(x, Wg, bg, W1, W3, W2):
    xf = x.reshape(S, D)
    pos, ws, meta, aux = _router(xf, Wg, bg.reshape(1, E))
    xs = _sc_dispatch()(xf, pos)
    ys = _experts(meta, xs, W1, W3, W2)
    out = _sc_combine()(ys, pos, ws)
    return (out.reshape(x.shape), aux[0, 0])
